# baseline (speedup 1.0000x reference)
import functools

import jax
import jax.numpy as jnp
import numpy as np
from jax import lax
from jax.experimental import pallas as pl
from jax.experimental.pallas import tpu as pltpu
from jax.experimental.pallas import tpu_sc as plsc

_NUM_BINS = 48
_D = 16
_NCAT = 26
_NNUM = 13
_VOCAB = 100000
_B = 16384
_H1, _H2 = 128, 64
_NR = _NNUM * (_NUM_BINS - 2)
_NRP = 608
_NC, _NS = 2, 16
_NW = _NC * _NS
_NFP = 32
_ROWS = _B * _NFP
_PER_W = _ROWS // _NW
_SUB = 128
_NSUB = 16
_CH = _SUB * _NSUB
_NCHUNK = _PER_W // _CH


def _sc_gather(table2d, idx3):
    mesh = plsc.VectorSubcoreMesh(core_axis_name="c", subcore_axis_name="s")

    @functools.partial(
        pl.kernel, mesh=mesh,
        compiler_params=pltpu.CompilerParams(use_tc_tiling_on_sc=False),
        out_type=jax.ShapeDtypeStruct((_ROWS, _D), jnp.float32),
        scratch_types=[
            pltpu.VMEM((_NSUB, _SUB), jnp.int32),
            pltpu.VMEM((_CH, _D), jnp.float32),
            pltpu.SemaphoreType.DMA,
        ],
    )
    def gather_k(table_hbm, idx_hbm, out_hbm, idx_v, rows_v, sem):
        wid = lax.axis_index("s") * _NC + lax.axis_index("c")

        def chunk(c, carry):
            pltpu.sync_copy(idx_hbm.at[wid * _NCHUNK + c], idx_v)
            descs = [
                pltpu.async_copy(table_hbm.at[idx_v.at[k]],
                                 rows_v.at[pl.ds(k * _SUB, _SUB)], sem)
                for k in range(_NSUB)
            ]
            for d_ in descs:
                d_.wait()
            pltpu.sync_copy(rows_v, out_hbm.at[pl.ds(wid * _PER_W + c * _CH, _CH)])
            return carry

        lax.fori_loop(0, _NCHUNK, chunk, 0)

    return gather_k(table2d, idx3)


def _mlp(num_p, gath, Ep, cvec, igvec, W1c, G, W1n, b1p, W2, b2, W3, b3):
    bm = 2048
    f32 = jnp.float32

    def body(num_ref, gath_ref, e_ref, c_ref, ig_ref, w1c_ref, g_ref, w1n_ref,
             b1_ref, w2_ref, b2_ref, w3_ref, b3_ref, out_ref):
        v = num_ref[...]
        vb = lax.dot(v, e_ref[...], precision=lax.Precision.HIGHEST)
        r = jnp.clip((vb - c_ref[...]) * ig_ref[...], 0.0, 1.0)
        g4 = gath_ref[...]
        h = jnp.dot(g4[:, 0].reshape(bm, 128), w1c_ref[0],
                    preferred_element_type=f32)
        for j in range(1, 4):
            h = h + jnp.dot(g4[:, j].reshape(bm, 128), w1c_ref[j],
                            preferred_element_type=f32)
        h = h + jnp.dot(r, g_ref[...], preferred_element_type=f32)
        h = h + jnp.dot(v, w1n_ref[...], preferred_element_type=f32)
        h = jnp.maximum(h + b1_ref[...], 0.0)
        h2 = jnp.maximum(
            jnp.dot(h, w2_ref[...], preferred_element_type=f32) + b2_ref[...], 0.0)
        out_ref[...] = jnp.dot(h2, w3_ref[...], preferred_element_type=f32) + b3_ref[...]

    return pl.pallas_call(
        body,
        grid=(_B // bm,),
        in_specs=[
            pl.BlockSpec((bm, 16), lambda i: (i, 0)),
            pl.BlockSpec((bm // 8, 4, 8, 128), lambda i: (i, 0, 0, 0)),
            pl.BlockSpec((16, _NRP), lambda i: (0, 0)),
            pl.BlockSpec((1, _NRP), lambda i: (0, 0)),
            pl.BlockSpec((1, _NRP), lambda i: (0, 0)),
            pl.BlockSpec((4, 128, _H1), lambda i: (0, 0, 0)),
            pl.BlockSpec((_NRP, _H1), lambda i: (0, 0)),
            pl.BlockSpec((16, _H1), lambda i: (0, 0)),
            pl.BlockSpec((1, _H1), lambda i: (0, 0)),
            pl.BlockSpec((_H1, _H2), lambda i: (0, 0)),
            pl.BlockSpec((1, _H2), lambda i: (0, 0)),
            pl.BlockSpec((_H2, 1), lambda i: (0, 0)),
            pl.BlockSpec((1, 1), lambda i: (0, 0)),
        ],
        out_specs=pl.BlockSpec((bm, 1), lambda i: (i, 0)),
        out_shape=jax.ShapeDtypeStruct((_B, 1), f32),
    )(num_p, gath, Ep, cvec, igvec, W1c, G, W1n, b1p, W2, b2, W3, b3)


def _e_matrix():
    e = np.zeros((16, _NRP), np.float32)
    for f in range(_NNUM):
        e[f, f * 46:(f + 1) * 46] = 1.0
    return jnp.asarray(e)


def kernel(numerical, categorical, cat_tables, num_tables, W1, b1, W2, b2, W3, b3):
    f32 = jnp.float32
    q = jnp.linspace(0.0, 1.0, _NUM_BINS + 1)[1:-1]
    bnd = (jnp.sqrt(2.0) * jax.scipy.special.erfinv(2.0 * q - 1.0)).astype(f32)
    c46 = bnd[:-1]
    ig46 = 1.0 / (bnd[1:] - bnd[:-1] + 1e-8)
    cvec = jnp.zeros((1, _NRP), f32).at[0, :_NR].set(jnp.tile(c46, _NNUM))
    igvec = jnp.zeros((1, _NRP), f32).at[0, :_NR].set(jnp.tile(ig46, _NNUM))
    Ep = _e_matrix()
    W1num = W1[_NNUM + _NCAT * _D:, :].reshape(_NNUM, _D, _H1)
    dT = num_tables[:, 2:, :] - num_tables[:, 1:-1, :]
    G = jnp.einsum('fkd,fdh->fkh', dT, W1num).reshape(_NR, _H1)
    G = jnp.concatenate([G, jnp.zeros((_NRP - _NR, _H1), f32)], axis=0)
    base = jnp.einsum('fd,fdh->h', num_tables[:, 1, :], W1num)
    b1p = (b1 + base).reshape(1, _H1)
    W1n = jnp.concatenate([W1[:_NNUM], jnp.zeros((3, _H1), f32)], axis=0)
    W1c = jnp.concatenate(
        [W1[_NNUM:_NNUM + _NCAT * _D],
         jnp.zeros(((_NFP - _NCAT) * _D, _H1), f32)], axis=0).reshape(4, 128, _H1)
    num_p = jnp.concatenate([numerical, jnp.zeros((_B, 3), f32)], axis=1)
    flat_idx = jnp.concatenate(
        [categorical.astype(jnp.int32)
         + (jnp.arange(_NCAT, dtype=jnp.int32) * _VOCAB)[None, :],
         jnp.zeros((_B, _NFP - _NCAT), jnp.int32)], axis=1)
    idx_perm = flat_idx.reshape(_B // 8, 8, 4, 8).transpose(0, 2, 1, 3)
    idx3 = idx_perm.reshape(_NW * _NCHUNK, _NSUB, _SUB)
    table2d = cat_tables.reshape(_NCAT * _VOCAB, _D)
    gath = _sc_gather(table2d, idx3).reshape(_B // 8, 4, 8, 128)
    return _mlp(num_p, gath, Ep, cvec, igvec, W1c, G, W1n, b1p,
                W2, b2.reshape(1, _H2), W3, b3.reshape(1, 1))


# speedup vs baseline: 1.3931x; 1.3931x over previous
import functools

import jax
import jax.numpy as jnp
import numpy as np
from jax import lax
from jax.experimental import pallas as pl
from jax.experimental.pallas import tpu as pltpu
from jax.experimental.pallas import tpu_sc as plsc

_NUM_BINS = 48
_D = 16
_NCAT = 26
_NNUM = 13
_VOCAB = 100000
_B = 16384
_H1, _H2 = 128, 64
_NR = _NNUM * (_NUM_BINS - 2)
_NRP = 608
_NC, _NS = 2, 16
_NW = _NC * _NS
_NFP = 32
_ROWS = _B * _NFP
_PER_W = _ROWS // _NW
_SUB = 128
_NSUB = 16
_CH = _SUB * _NSUB
_NCHUNK = _PER_W // _CH


def _sc_gather(table2d, idx3):
    mesh = plsc.VectorSubcoreMesh(core_axis_name="c", subcore_axis_name="s")

    @functools.partial(
        pl.kernel, mesh=mesh,
        compiler_params=pltpu.CompilerParams(use_tc_tiling_on_sc=False),
        out_type=jax.ShapeDtypeStruct((_ROWS, _D), jnp.float32),
        scratch_types=[
            pltpu.VMEM((_NSUB, _SUB), jnp.int32),
            pltpu.VMEM((_CH, _D), jnp.float32),
            pltpu.SemaphoreType.DMA,
        ],
    )
    def gather_k(table_hbm, idx_hbm, out_hbm, idx_v, rows_v, sem):
        wid = lax.axis_index("s") * _NC + lax.axis_index("c")

        def chunk(c, carry):
            pltpu.sync_copy(idx_hbm.at[wid * _NCHUNK + c], idx_v)
            descs = [
                pltpu.async_copy(table_hbm.at[idx_v.at[k]],
                                 rows_v.at[pl.ds(k * _SUB, _SUB)], sem)
                for k in range(_NSUB)
            ]
            for d_ in descs:
                d_.wait()
            pltpu.sync_copy(rows_v, out_hbm.at[pl.ds(wid * _PER_W + c * _CH, _CH)])
            return carry

        lax.fori_loop(0, _NCHUNK, chunk, 0)

    return gather_k(table2d, idx3)


def _mlp(num_p, gath, Ep, cvec, igvec, W1c, G, W1n, b1p, W2, b2, W3, b3):
    bm = 2048
    f32 = jnp.float32

    def body(num_ref, gath_ref, e_ref, c_ref, ig_ref, w1c_ref, g_ref, w1n_ref,
             b1_ref, w2_ref, b2_ref, w3_ref, b3_ref, out_ref):
        v = num_ref[...]
        vb = lax.dot(v, e_ref[...], precision=lax.Precision.HIGHEST)
        r = jnp.clip((vb - c_ref[...]) * ig_ref[...], 0.0, 1.0)
        g4 = gath_ref[...]
        h = jnp.dot(g4[:, 0].reshape(bm, 128), w1c_ref[0],
                    preferred_element_type=f32)
        for j in range(1, 4):
            h = h + jnp.dot(g4[:, j].reshape(bm, 128), w1c_ref[j],
                            preferred_element_type=f32)
        h = h + jnp.dot(r, g_ref[...], preferred_element_type=f32)
        h = h + jnp.dot(v, w1n_ref[...], preferred_element_type=f32)
        h = jnp.maximum(h + b1_ref[...], 0.0)
        h2 = jnp.maximum(
            jnp.dot(h, w2_ref[...], preferred_element_type=f32) + b2_ref[...], 0.0)
        out_ref[...] = jnp.dot(h2, w3_ref[...], preferred_element_type=f32) + b3_ref[...]

    return pl.pallas_call(
        body,
        grid=(_B // bm,),
        in_specs=[
            pl.BlockSpec((bm, 16), lambda i: (i, 0)),
            pl.BlockSpec((bm // 8, 4, 8, 128), lambda i: (i, 0, 0, 0)),
            pl.BlockSpec((16, _NRP), lambda i: (0, 0)),
            pl.BlockSpec((1, _NRP), lambda i: (0, 0)),
            pl.BlockSpec((1, _NRP), lambda i: (0, 0)),
            pl.BlockSpec((4, 128, _H1), lambda i: (0, 0, 0)),
            pl.BlockSpec((_NRP, _H1), lambda i: (0, 0)),
            pl.BlockSpec((16, _H1), lambda i: (0, 0)),
            pl.BlockSpec((1, _H1), lambda i: (0, 0)),
            pl.BlockSpec((_H1, _H2), lambda i: (0, 0)),
            pl.BlockSpec((1, _H2), lambda i: (0, 0)),
            pl.BlockSpec((_H2, 1), lambda i: (0, 0)),
            pl.BlockSpec((1, 1), lambda i: (0, 0)),
        ],
        out_specs=pl.BlockSpec((bm, 1), lambda i: (i, 0)),
        out_shape=jax.ShapeDtypeStruct((_B, 1), f32),
    )(num_p, gath, Ep, cvec, igvec, W1c, G, W1n, b1p, W2, b2, W3, b3)


def _e_matrix():
    e = np.zeros((16, _NRP), np.float32)
    for f in range(_NNUM):
        e[f, f * 46:(f + 1) * 46] = 1.0
    return jnp.asarray(e)


def kernel(numerical, categorical, cat_tables, num_tables, W1, b1, W2, b2, W3, b3):
    f32 = jnp.float32
    q = jnp.linspace(0.0, 1.0, _NUM_BINS + 1)[1:-1]
    bnd = (jnp.sqrt(2.0) * jax.scipy.special.erfinv(2.0 * q - 1.0)).astype(f32)
    c46 = bnd[:-1]
    ig46 = 1.0 / (bnd[1:] - bnd[:-1] + 1e-8)
    cvec = jnp.zeros((1, _NRP), f32).at[0, :_NR].set(jnp.tile(c46, _NNUM))
    igvec = jnp.zeros((1, _NRP), f32).at[0, :_NR].set(jnp.tile(ig46, _NNUM))
    Ep = _e_matrix()
    W1num = W1[_NNUM + _NCAT * _D:, :].reshape(_NNUM, _D, _H1)
    dT = num_tables[:, 2:, :] - num_tables[:, 1:-1, :]
    G = jnp.einsum('fkd,fdh->fkh', dT, W1num).reshape(_NR, _H1)
    G = jnp.concatenate([G, jnp.zeros((_NRP - _NR, _H1), f32)], axis=0)
    base = jnp.einsum('fd,fdh->h', num_tables[:, 1, :], W1num)
    b1p = (b1 + base).reshape(1, _H1)
    W1n = jnp.concatenate([W1[:_NNUM], jnp.zeros((3, _H1), f32)], axis=0)
    W1c = jnp.concatenate(
        [W1[_NNUM:_NNUM + _NCAT * _D],
         jnp.zeros(((_NFP - _NCAT) * _D, _H1), f32)], axis=0).reshape(4, 128, _H1)
    num_p = jnp.concatenate([numerical, jnp.zeros((_B, 3), f32)], axis=1)
    pad_idx = (jnp.arange(_B, dtype=jnp.int32)[:, None] * (_NFP - _NCAT)
               + jnp.arange(_NFP - _NCAT, dtype=jnp.int32)[None, :])
    flat_idx = jnp.concatenate(
        [categorical.astype(jnp.int32)
         + (jnp.arange(_NCAT, dtype=jnp.int32) * _VOCAB)[None, :],
         pad_idx], axis=1)
    idx_perm = flat_idx.reshape(_B // 8, 8, 4, 8).transpose(0, 2, 1, 3)
    idx3 = idx_perm.reshape(_NW * _NCHUNK, _NSUB, _SUB)
    table2d = cat_tables.reshape(_NCAT * _VOCAB, _D)
    gath = _sc_gather(table2d, idx3).reshape(_B // 8, 4, 8, 128)
    return _mlp(num_p, gath, Ep, cvec, igvec, W1c, G, W1n, b1p,
                W2, b2.reshape(1, _H2), W3, b3.reshape(1, 1))


# speedup vs baseline: 1.4042x; 1.0080x over previous
import functools

import jax
import jax.numpy as jnp
import numpy as np
from jax import lax
from jax.experimental import pallas as pl
from jax.experimental.pallas import tpu as pltpu
from jax.experimental.pallas import tpu_sc as plsc

_NUM_BINS = 48
_D = 16
_NCAT = 26
_NNUM = 13
_VOCAB = 100000
_B = 16384
_H1, _H2 = 128, 64
_NR = _NNUM * (_NUM_BINS - 2)
_NRP = 608
_NC, _NS = 2, 16
_NW = _NC * _NS
_NFP = 32
_ROWS = _B * _NFP
_PER_W = _ROWS // _NW
_SUB = 128
_NSUB = 16
_CH = _SUB * _NSUB
_NCHUNK = _PER_W // _CH


def _sc_gather(table2d, categorical):
    mesh = plsc.VectorSubcoreMesh(core_axis_name="c", subcore_axis_name="s")

    @functools.partial(
        pl.kernel, mesh=mesh,
        compiler_params=pltpu.CompilerParams(use_tc_tiling_on_sc=False,
                                             needs_layout_passes=False),
        out_type=jax.ShapeDtypeStruct((_ROWS, _D), jnp.float32),
        scratch_types=[
            pltpu.VMEM((64, _NCAT), jnp.int32),
            pltpu.VMEM((_NSUB, _SUB), jnp.int32),
            pltpu.VMEM((_CH, _D), jnp.float32),
            pltpu.SemaphoreType.DMA,
        ],
    )
    def gather_k(table_hbm, cat_hbm, out_hbm, nat_v, idx_v, rows_v, sem):
        wid = lax.axis_index("s") * _NC + lax.axis_index("c")
        i32 = jnp.int32
        lane = lax.iota(i32, 16)
        s_pat = lax.shift_right_logical(lane, 3)
        f_pat = lax.bitwise_and(lane, 7)

        def chunk(c, carry):
            b0 = wid * 512 + c * 64
            pltpu.sync_copy(cat_hbm.at[pl.ds(b0, 64)], nat_v)

            def permute(n, carry2):
                i_loc = lax.shift_right_logical(n, 4)
                j = lax.bitwise_and(lax.shift_right_logical(n, 2), 3)
                s_hi = lax.bitwise_and(n, 3) * 2
                row = jnp.full((16,), 8 * i_loc + s_hi, i32) + s_pat
                col = jnp.full((16,), 8 * j, i32) + f_pat
                valid = col < _NCAT
                col_c = jnp.minimum(col, _NCAT - 1)
                g = plsc.load_gather(nat_v, [row, col_c]) + col * _VOCAB
                bvec = row + b0
                padv = bvec * (_NFP - _NCAT) + (col - _NCAT)
                val = jnp.where(valid, g, padv)
                idx_v[lax.shift_right_logical(n, 3),
                      pl.ds(lax.bitwise_and(n, 7) * 16, 16)] = val
                return carry2

            lax.fori_loop(0, 128, permute, 0)
            descs = [
                pltpu.async_copy(table_hbm.at[idx_v.at[k]],
                                 rows_v.at[pl.ds(k * _SUB, _SUB)], sem)
                for k in range(_NSUB)
            ]
            for d_ in descs:
                d_.wait()
            pltpu.sync_copy(rows_v, out_hbm.at[pl.ds(wid * _PER_W + c * _CH, _CH)])
            return carry

        lax.fori_loop(0, _NCHUNK, chunk, 0)

    return gather_k(table2d, categorical)


def _mlp(num_p, gath, Ep, cvec, igvec, W1c, G, W1n, b1p, W2, b2, W3, b3):
    bm = 2048
    f32 = jnp.float32

    def body(num_ref, gath_ref, e_ref, c_ref, ig_ref, w1c_ref, g_ref, w1n_ref,
             b1_ref, w2_ref, b2_ref, w3_ref, b3_ref, out_ref):
        v = num_ref[...]
        vb = lax.dot(v, e_ref[...], precision=lax.Precision.HIGHEST)
        r = jnp.clip((vb - c_ref[...]) * ig_ref[...], 0.0, 1.0)
        g4 = gath_ref[...]
        h = jnp.dot(g4[:, 0].reshape(bm, 128), w1c_ref[0],
                    preferred_element_type=f32)
        for j in range(1, 4):
            h = h + jnp.dot(g4[:, j].reshape(bm, 128), w1c_ref[j],
                            preferred_element_type=f32)
        h = h + jnp.dot(r, g_ref[...], preferred_element_type=f32)
        h = h + jnp.dot(v, w1n_ref[...], preferred_element_type=f32)
        h = jnp.maximum(h + b1_ref[...], 0.0)
        h2 = jnp.maximum(
            jnp.dot(h, w2_ref[...], preferred_element_type=f32) + b2_ref[...], 0.0)
        out_ref[...] = jnp.dot(h2, w3_ref[...], preferred_element_type=f32) + b3_ref[...]

    return pl.pallas_call(
        body,
        grid=(_B // bm,),
        in_specs=[
            pl.BlockSpec((bm, 16), lambda i: (i, 0)),
            pl.BlockSpec((bm // 8, 4, 8, 128), lambda i: (i, 0, 0, 0)),
            pl.BlockSpec((16, _NRP), lambda i: (0, 0)),
            pl.BlockSpec((1, _NRP), lambda i: (0, 0)),
            pl.BlockSpec((1, _NRP), lambda i: (0, 0)),
            pl.BlockSpec((4, 128, _H1), lambda i: (0, 0, 0)),
            pl.BlockSpec((_NRP, _H1), lambda i: (0, 0)),
            pl.BlockSpec((16, _H1), lambda i: (0, 0)),
            pl.BlockSpec((1, _H1), lambda i: (0, 0)),
            pl.BlockSpec((_H1, _H2), lambda i: (0, 0)),
            pl.BlockSpec((1, _H2), lambda i: (0, 0)),
            pl.BlockSpec((_H2, 1), lambda i: (0, 0)),
            pl.BlockSpec((1, 1), lambda i: (0, 0)),
        ],
        out_specs=pl.BlockSpec((bm, 1), lambda i: (i, 0)),
        out_shape=jax.ShapeDtypeStruct((_B, 1), f32),
    )(num_p, gath, Ep, cvec, igvec, W1c, G, W1n, b1p, W2, b2, W3, b3)


def _e_matrix():
    e = np.zeros((16, _NRP), np.float32)
    for f in range(_NNUM):
        e[f, f * 46:(f + 1) * 46] = 1.0
    return jnp.asarray(e)


def kernel(numerical, categorical, cat_tables, num_tables, W1, b1, W2, b2, W3, b3):
    f32 = jnp.float32
    q = jnp.linspace(0.0, 1.0, _NUM_BINS + 1)[1:-1]
    bnd = (jnp.sqrt(2.0) * jax.scipy.special.erfinv(2.0 * q - 1.0)).astype(f32)
    c46 = bnd[:-1]
    ig46 = 1.0 / (bnd[1:] - bnd[:-1] + 1e-8)
    cvec = jnp.zeros((1, _NRP), f32).at[0, :_NR].set(jnp.tile(c46, _NNUM))
    igvec = jnp.zeros((1, _NRP), f32).at[0, :_NR].set(jnp.tile(ig46, _NNUM))
    Ep = _e_matrix()
    W1num = W1[_NNUM + _NCAT * _D:, :].reshape(_NNUM, _D, _H1)
    dT = num_tables[:, 2:, :] - num_tables[:, 1:-1, :]
    G = jnp.einsum('fkd,fdh->fkh', dT, W1num).reshape(_NR, _H1)
    G = jnp.concatenate([G, jnp.zeros((_NRP - _NR, _H1), f32)], axis=0)
    base = jnp.einsum('fd,fdh->h', num_tables[:, 1, :], W1num)
    b1p = (b1 + base).reshape(1, _H1)
    W1n = jnp.concatenate([W1[:_NNUM], jnp.zeros((3, _H1), f32)], axis=0)
    W1c = jnp.concatenate(
        [W1[_NNUM:_NNUM + _NCAT * _D],
         jnp.zeros(((_NFP - _NCAT) * _D, _H1), f32)], axis=0).reshape(4, 128, _H1)
    num_p = jnp.concatenate([numerical, jnp.zeros((_B, 3), f32)], axis=1)
    table2d = cat_tables.reshape(_NCAT * _VOCAB, _D)
    gath = _sc_gather(table2d, categorical.astype(jnp.int32)
                      ).reshape(_B // 8, 4, 8, 128)
    return _mlp(num_p, gath, Ep, cvec, igvec, W1c, G, W1n, b1p,
                W2, b2.reshape(1, _H2), W3, b3.reshape(1, 1))
